# baseline (speedup 1.0000x reference)
import functools

import jax
import jax.numpy as jnp
from jax import lax
from jax.experimental import pallas as pl
from jax.experimental.pallas import tpu as pltpu
from jax.experimental.pallas import tpu_sc as plsc

_IDX_CHUNK = 128


def _gather_kernel(B, V, D, b_per_w, n_chunk, NC):
    mesh = plsc.VectorSubcoreMesh(core_axis_name="c", subcore_axis_name="s")

    @functools.partial(
        pl.kernel,
        mesh=mesh,
        out_type=jax.ShapeDtypeStruct((B, D), jnp.float32),
        scratch_types=[
            pltpu.VMEM((n_chunk, _IDX_CHUNK), jnp.int32),
            pltpu.VMEM((b_per_w, D), jnp.float32),
            pltpu.SemaphoreType.DMA((n_chunk,)),
            pltpu.SemaphoreType.DMA,
        ],
    )
    def k(table_hbm, idx_hbm, out_hbm, idx_v, rows_v, gsem, wsem):
        wid = lax.axis_index("s") * NC + lax.axis_index("c")
        base = wid * b_per_w
        pltpu.sync_copy(idx_hbm.at[wid], idx_v)
        gathers = []
        for j in range(n_chunk):
            gathers.append(
                pltpu.async_copy(
                    table_hbm.at[idx_v.at[j]],
                    rows_v.at[pl.ds(j * _IDX_CHUNK, _IDX_CHUNK)],
                    gsem.at[j],
                )
            )
        writes = []
        for j in range(n_chunk):
            gathers[j].wait()
            writes.append(
                pltpu.async_copy(
                    rows_v.at[pl.ds(j * _IDX_CHUNK, _IDX_CHUNK)],
                    out_hbm.at[pl.ds(base + j * _IDX_CHUNK, _IDX_CHUNK)],
                    wsem,
                )
            )
        for c in writes:
            c.wait()

    return k


def kernel(embeddings, t):
    V, D = embeddings.shape
    (B,) = t.shape
    info = plsc.get_sparse_core_info()
    NC, NS = info.num_cores, info.num_subcores
    NW = NC * NS
    b_per_w = B // NW
    n_chunk = b_per_w // _IDX_CHUNK
    k = _gather_kernel(B, V, D, b_per_w, n_chunk, NC)
    t_blocked = t.reshape(NW, n_chunk, _IDX_CHUNK)
    return k(embeddings, t_blocked)


# speedup vs baseline: 1.0033x; 1.0033x over previous
import functools

import jax
import jax.numpy as jnp
from jax import lax
from jax.experimental import pallas as pl
from jax.experimental.pallas import tpu as pltpu
from jax.experimental.pallas import tpu_sc as plsc

_IDX_CHUNK = 128


def _gather_kernel(B, V, D, b_per_w, n_chunk, NC):
    mesh = plsc.VectorSubcoreMesh(core_axis_name="c", subcore_axis_name="s")

    @functools.partial(
        pl.kernel,
        mesh=mesh,
        out_type=jax.ShapeDtypeStruct((B, D), jnp.float32),
        scratch_types=[
            pltpu.VMEM((n_chunk, _IDX_CHUNK), jnp.int32),
            pltpu.VMEM((b_per_w, D), jnp.float32),
            pltpu.SemaphoreType.DMA((n_chunk,)),
            pltpu.SemaphoreType.DMA((n_chunk,)),
            pltpu.SemaphoreType.DMA,
        ],
    )
    def k(table_hbm, idx_hbm, out_hbm, idx_v, rows_v, isem, gsem, wsem):
        wid = lax.axis_index("s") * NC + lax.axis_index("c")
        base = wid * b_per_w
        idx_copies = []
        for j in range(n_chunk):
            idx_copies.append(
                pltpu.async_copy(
                    idx_hbm.at[pl.ds(base + j * _IDX_CHUNK, _IDX_CHUNK)],
                    idx_v.at[j],
                    isem.at[j],
                )
            )
        gathers = []
        for j in range(n_chunk):
            idx_copies[j].wait()
            gathers.append(
                pltpu.async_copy(
                    table_hbm.at[idx_v.at[j]],
                    rows_v.at[pl.ds(j * _IDX_CHUNK, _IDX_CHUNK)],
                    gsem.at[j],
                )
            )
        writes = []
        for j in range(n_chunk):
            gathers[j].wait()
            writes.append(
                pltpu.async_copy(
                    rows_v.at[pl.ds(j * _IDX_CHUNK, _IDX_CHUNK)],
                    out_hbm.at[pl.ds(base + j * _IDX_CHUNK, _IDX_CHUNK)],
                    wsem,
                )
            )
        for c in writes:
            c.wait()

    return k


def kernel(embeddings, t):
    V, D = embeddings.shape
    (B,) = t.shape
    info = plsc.get_sparse_core_info()
    NC, NS = info.num_cores, info.num_subcores
    NW = NC * NS
    b_per_w = B // NW
    n_chunk = b_per_w // _IDX_CHUNK
    k = _gather_kernel(B, V, D, b_per_w, n_chunk, NC)
    return k(embeddings, t)


# speedup vs baseline: 1.0067x; 1.0034x over previous
import functools

import jax
import jax.numpy as jnp
from jax import lax
from jax.experimental import pallas as pl
from jax.experimental.pallas import tpu as pltpu
from jax.experimental.pallas import tpu_sc as plsc

_IDX_CHUNK = 128


def _gather_kernel(B, V, D, b_per_w, n_chunk, NC):
    mesh = plsc.VectorSubcoreMesh(core_axis_name="c", subcore_axis_name="s")

    @functools.partial(
        pl.kernel,
        mesh=mesh,
        out_type=jax.ShapeDtypeStruct((B, D), jnp.float32),
        scratch_types=[
            pltpu.VMEM((b_per_w,), jnp.int32),
            pltpu.VMEM((b_per_w, D), jnp.float32),
            pltpu.SemaphoreType.DMA,
        ],
    )
    def k(table_hbm, idx_hbm, out_hbm, idx_v, rows_v, sem):
        wid = lax.axis_index("s") * NC + lax.axis_index("c")
        base = wid * b_per_w
        pltpu.sync_copy(idx_hbm.at[pl.ds(base, b_per_w)], idx_v)
        pltpu.async_copy(table_hbm.at[idx_v], rows_v, sem).wait()
        pltpu.sync_copy(rows_v, out_hbm.at[pl.ds(base, b_per_w)])

    return k


def kernel(embeddings, t):
    V, D = embeddings.shape
    (B,) = t.shape
    info = plsc.get_sparse_core_info()
    NC, NS = info.num_cores, info.num_subcores
    NW = NC * NS
    b_per_w = B // NW
    n_chunk = b_per_w // _IDX_CHUNK
    k = _gather_kernel(B, V, D, b_per_w, n_chunk, NC)
    return k(embeddings, t)
